# baseline (speedup 1.0000x reference)
import functools

import jax
import jax.numpy as jnp
from jax import lax
from jax.experimental import pallas as pl
from jax.experimental.pallas import tpu as pltpu
from jax.experimental.pallas import tpu_sc as plsc

N = 10000
E = 320000
D_IN = 128
D_HID = 128
D_OUT = 64

NC = 2
NS = 16
LANES = 16
CHUNK = 128
CPT = 160
EPT = CPT * CHUNK
NPAD = 10016
RPT = N // NS
RPT_PAD = 640

_mesh = plsc.VectorSubcoreMesh(core_axis_name="c", subcore_axis_name="s")


def _deg_body(dst_hbm, out_hbm, deg_v, dst_v):
    c = lax.axis_index("c")
    s = lax.axis_index("s")

    zeros16 = jnp.zeros((LANES,), jnp.float32)

    def zero_body(i, carry):
        deg_v[pl.ds(i * LANES, LANES)] = zeros16
        return carry

    lax.fori_loop(0, NPAD // LANES, zero_body, 0)

    pltpu.sync_copy(dst_hbm.at[s], dst_v)

    ones16 = jnp.ones((LANES,), jnp.float32)
    half = CPT // 2

    def edge_body(j, carry):
        for k in range(CHUNK // LANES):
            idx = dst_v[j, pl.ds(k * LANES, LANES)]
            plsc.addupdate_scatter(deg_v, [idx], ones16)
        return carry

    lax.fori_loop(c * half, (c + 1) * half, edge_body, 0)

    wid = s * NC + c
    pltpu.sync_copy(deg_v, out_hbm.at[wid])


_deg = functools.partial(
    pl.kernel,
    out_type=jax.ShapeDtypeStruct((NC * NS, NPAD), jnp.float32),
    mesh=_mesh,
    scratch_types=[
        pltpu.VMEM((NPAD,), jnp.float32),
        pltpu.VMEM((CPT, CHUNK), jnp.int32),
    ],
    compiler_params=pltpu.CompilerParams(needs_layout_passes=False, use_tc_tiling_on_sc=False),
)(_deg_body)


def _make_agg(dh, nbuf, lead, ph, finalize=False):
    w = CPT // ph
    assert w % nbuf == 0 and lead < nbuf

    def agg_body(hs_hbm, src_hbm, dst_hbm, *rest):
        if finalize:
            (dis_hbm, b_hbm, out_hbm, hs_sh, acc_sh, src_v, dst_v, buf,
             sem, ssem, dis_v, b_v, stage) = rest
        else:
            (out_hbm, hs_sh, acc_sh, src_v, dst_v, buf, sem, ssem) = rest
        c = lax.axis_index("c")
        s = lax.axis_index("s")
        rs = s * RPT

        pltpu.sync_copy(hs_hbm.at[c, pl.ds(rs, RPT)], hs_sh.at[pl.ds(rs, RPT)])
        pltpu.sync_copy(hs_hbm.at[c, pl.ds(rs, RPT)], acc_sh.at[pl.ds(rs, RPT)])
        plsc.subcore_barrier()

        for p in range(ph):
            pltpu.sync_copy(src_hbm.at[s, pl.ds(p * w, w)], src_v)
            pltpu.sync_copy(dst_hbm.at[s, pl.ds(p * w, w)], dst_v)

            for b in range(lead):
                pltpu.async_copy(hs_sh.at[src_v.at[b]], buf.at[b],
                                 sem.at[b])

            def outer_body(i, carry):
                j0 = i * nbuf
                for b in range(nbuf):
                    j = j0 + b
                    pltpu.make_async_copy(
                        hs_sh.at[src_v.at[j]], buf.at[b], sem.at[b]).wait()
                    pltpu.async_copy(buf.at[b], acc_sh.at[dst_v.at[j]],
                                     ssem.at[b], add=True)
                    f = j + lead
                    bf = (b + lead) % nbuf

                    @pl.when(f < w)
                    def _():
                        @pl.when(j >= nbuf - lead)
                        def _():
                            pltpu.make_async_copy(
                                buf.at[bf], acc_sh.at[dst_v.at[j]],
                                ssem.at[bf]).wait()

                        pltpu.async_copy(hs_sh.at[src_v.at[f]], buf.at[bf],
                                         sem.at[bf])
                return carry

            lax.fori_loop(0, w // nbuf, outer_body, 0)

            for b in range(nbuf):
                pltpu.make_async_copy(
                    buf.at[b], acc_sh.at[dst_v.at[w - nbuf + b]],
                    ssem.at[b]).wait()

        plsc.subcore_barrier()

        if not finalize:
            pltpu.sync_copy(acc_sh.at[pl.ds(rs, RPT)],
                            out_hbm.at[c, pl.ds(rs, RPT)])
        else:
            pltpu.sync_copy(acc_sh.at[pl.ds(rs, RPT)], stage)
            pltpu.sync_copy(dis_hbm.at[s], dis_v)
            pltpu.sync_copy(b_hbm.at[c], b_v)
            bias_vecs = [b_v[pl.ds(k * LANES, LANES)]
                         for k in range(dh // LANES)]

            def row_body(r, carry):
                d = plsc.load_gather(dis_v, [jnp.zeros((LANES,), jnp.int32) + r])
                for k in range(dh // LANES):
                    v = stage[r, pl.ds(k * LANES, LANES)]
                    stage[r, pl.ds(k * LANES, LANES)] = v * d + bias_vecs[k]
                return carry

            lax.fori_loop(0, RPT, row_body, 0)
            pltpu.sync_copy(
                stage, out_hbm.at[pl.ds(rs, RPT), pl.ds(c * dh, dh)])

    scratch = [
        pltpu.VMEM_SHARED((N, dh), jnp.float32),
        pltpu.VMEM_SHARED((NPAD, dh), jnp.float32),
        pltpu.VMEM((w, CHUNK), jnp.int32),
        pltpu.VMEM((w, CHUNK), jnp.int32),
        pltpu.VMEM((nbuf, CHUNK, dh), jnp.float32),
        pltpu.SemaphoreType.DMA((nbuf,)),
        pltpu.SemaphoreType.DMA((nbuf,)),
    ]
    if finalize:
        scratch += [
            pltpu.VMEM((RPT_PAD,), jnp.float32),
            pltpu.VMEM((dh,), jnp.float32),
            pltpu.VMEM((RPT, dh), jnp.float32),
        ]
    out_shape = (N, NC * dh) if finalize else (NC, N, dh)
    return functools.partial(
        pl.kernel,
        out_type=jax.ShapeDtypeStruct(out_shape, jnp.float32),
        mesh=_mesh,
        scratch_types=scratch,
        compiler_params=pltpu.CompilerParams(needs_layout_passes=False, use_tc_tiling_on_sc=False),
    )(agg_body)


_agg_hid = _make_agg(D_HID // NC, nbuf=5, lead=3, ph=8)
_agg_out = _make_agg(D_OUT // NC, nbuf=8, lead=4, ph=2, finalize=True)


def _prep_body(degt_ref, x_ref, w1_ref, hs_ref, dis_ref):
    deg = jnp.sum(degt_ref[...], axis=1, keepdims=True) + 1.0
    dis = lax.rsqrt(deg)
    h = jnp.dot(x_ref[...], w1_ref[...], preferred_element_type=jnp.float32)
    hs = h * dis
    hs_ref[0] = hs[:, : D_HID // 2]
    hs_ref[1] = hs[:, D_HID // 2:]
    dis_ref[...] = dis


def _mid_body(agg_ref, dis_ref, b1_ref, w2_ref, out_ref):
    dis = dis_ref[...]
    h0 = jnp.maximum(agg_ref[0] * dis + b1_ref[0, : D_HID // 2], 0.0)
    h1 = jnp.maximum(agg_ref[1] * dis + b1_ref[0, D_HID // 2:], 0.0)
    hs2 = jnp.dot(h0, w2_ref[: D_HID // 2], preferred_element_type=jnp.float32)
    hs2 = hs2 + jnp.dot(h1, w2_ref[D_HID // 2:], preferred_element_type=jnp.float32)
    hs2 = hs2 * dis
    out_ref[0] = hs2[:, : D_OUT // 2]
    out_ref[1] = hs2[:, D_OUT // 2:]


_prep = pl.pallas_call(
    _prep_body,
    out_shape=[
        jax.ShapeDtypeStruct((NC, N, D_HID // 2), jnp.float32),
        jax.ShapeDtypeStruct((N, 1), jnp.float32),
    ],
)

_mid = pl.pallas_call(
    _mid_body,
    out_shape=jax.ShapeDtypeStruct((NC, N, D_OUT // 2), jnp.float32),
)

@jax.jit
def kernel(x, edge_index, W1, b1, W2, b2):
    src = edge_index[0].astype(jnp.int32)
    dst = edge_index[1].astype(jnp.int32)
    pad = NS * EPT - E
    src_p = jnp.concatenate([src, jnp.zeros((pad,), jnp.int32)]).reshape(NS, CPT, CHUNK)
    dst_p = jnp.concatenate([dst, jnp.full((pad,), N, jnp.int32)]).reshape(NS, CPT, CHUNK)

    deg_parts = _deg(dst_p)
    degt = deg_parts.T[:N]
    hs1, dis = _prep(degt, x, W1)
    agg1 = _agg_hid(hs1, src_p, dst_p)
    hs2 = _mid(agg1, dis, b1.reshape(1, -1), W2)
    dis_t = jnp.pad(dis.reshape(NS, RPT), ((0, 0), (0, RPT_PAD - RPT)))
    b2_t = b2.reshape(NC, D_OUT // NC)
    return _agg_out(hs2, src_p, dst_p, dis_t, b2_t)


# speedup vs baseline: 1.0318x; 1.0318x over previous
import functools

import jax
import jax.numpy as jnp
from jax import lax
from jax.experimental import pallas as pl
from jax.experimental.pallas import tpu as pltpu
from jax.experimental.pallas import tpu_sc as plsc

N = 10000
E = 320000
D_IN = 128
D_HID = 128
D_OUT = 64

NC = 2
NS = 16
LANES = 16
CHUNK = 128
CPT = 160
EPT = CPT * CHUNK
NPAD = 10016
RPT = N // NS
RPT_PAD = 640

_mesh = plsc.VectorSubcoreMesh(core_axis_name="c", subcore_axis_name="s")


def _deg_body(dst_hbm, out_hbm, deg_v, dst_v):
    c = lax.axis_index("c")
    s = lax.axis_index("s")

    zeros16 = jnp.zeros((LANES,), jnp.float32)

    def zero_body(i, carry):
        deg_v[pl.ds(i * LANES, LANES)] = zeros16
        return carry

    lax.fori_loop(0, NPAD // LANES, zero_body, 0)

    pltpu.sync_copy(dst_hbm.at[s], dst_v)

    ones16 = jnp.ones((LANES,), jnp.float32)
    half = CPT // 2

    def edge_body(j, carry):
        for k in range(CHUNK // LANES):
            idx = dst_v[j, pl.ds(k * LANES, LANES)]
            plsc.addupdate_scatter(deg_v, [idx], ones16)
        return carry

    lax.fori_loop(c * half, (c + 1) * half, edge_body, 0)

    wid = s * NC + c
    pltpu.sync_copy(deg_v, out_hbm.at[wid])


_deg = functools.partial(
    pl.kernel,
    out_type=jax.ShapeDtypeStruct((NC * NS, NPAD), jnp.float32),
    mesh=_mesh,
    scratch_types=[
        pltpu.VMEM((NPAD,), jnp.float32),
        pltpu.VMEM((CPT, CHUNK), jnp.int32),
    ],
    compiler_params=pltpu.CompilerParams(needs_layout_passes=False, use_tc_tiling_on_sc=False),
)(_deg_body)


def _make_agg(dh, nbuf, lead, ph, finalize=False):
    w = CPT // ph
    assert w % nbuf == 0 and lead < nbuf

    def agg_body(hs_hbm, src_hbm, dst_hbm, *rest):
        if finalize:
            (dis_hbm, b_hbm, out_hbm, hs_sh, acc_sh, src_v, dst_v, buf,
             sem, ssem, dis_v, b_v, stage) = rest
        else:
            (out_hbm, hs_sh, acc_sh, src_v, dst_v, buf, sem, ssem) = rest
        c = lax.axis_index("c")
        s = lax.axis_index("s")
        rs = s * RPT

        pltpu.sync_copy(hs_hbm.at[c, pl.ds(rs, RPT)], hs_sh.at[pl.ds(rs, RPT)])
        pltpu.sync_copy(hs_hbm.at[c, pl.ds(rs, RPT)], acc_sh.at[pl.ds(rs, RPT)])
        plsc.subcore_barrier()

        for p in range(ph):
            pltpu.sync_copy(src_hbm.at[s, pl.ds(p * w, w)], src_v)
            pltpu.sync_copy(dst_hbm.at[s, pl.ds(p * w, w)], dst_v)

            for b in range(lead):
                pltpu.async_copy(hs_sh.at[src_v.at[b]], buf.at[b],
                                 sem.at[b])

            def outer_body(i, carry):
                j0 = i * nbuf
                for b in range(nbuf):
                    j = j0 + b
                    pltpu.make_async_copy(
                        hs_sh.at[src_v.at[j]], buf.at[b], sem.at[b]).wait()
                    pltpu.async_copy(buf.at[b], acc_sh.at[dst_v.at[j]],
                                     ssem.at[b], add=True)
                    f = j + lead
                    bf = (b + lead) % nbuf

                    @pl.when(f < w)
                    def _():
                        @pl.when(j >= nbuf - lead)
                        def _():
                            pltpu.make_async_copy(
                                buf.at[bf], acc_sh.at[dst_v.at[j]],
                                ssem.at[bf]).wait()

                        pltpu.async_copy(hs_sh.at[src_v.at[f]], buf.at[bf],
                                         sem.at[bf])
                return carry

            lax.fori_loop(0, w // nbuf, outer_body, 0)

            for b in range(nbuf):
                pltpu.make_async_copy(
                    buf.at[b], acc_sh.at[dst_v.at[w - nbuf + b]],
                    ssem.at[b]).wait()

        plsc.subcore_barrier()

        if not finalize:
            pltpu.sync_copy(acc_sh.at[pl.ds(rs, RPT)],
                            out_hbm.at[c, pl.ds(rs, RPT)])
        else:
            pltpu.sync_copy(acc_sh.at[pl.ds(rs, RPT)], stage)
            pltpu.sync_copy(dis_hbm.at[s], dis_v)
            pltpu.sync_copy(b_hbm.at[c], b_v)
            bias_vecs = [b_v[pl.ds(k * LANES, LANES)]
                         for k in range(dh // LANES)]

            def row_body(r, carry):
                d = plsc.load_gather(dis_v, [jnp.zeros((LANES,), jnp.int32) + r])
                for k in range(dh // LANES):
                    v = stage[r, pl.ds(k * LANES, LANES)]
                    stage[r, pl.ds(k * LANES, LANES)] = v * d + bias_vecs[k]
                return carry

            lax.fori_loop(0, RPT, row_body, 0)
            pltpu.sync_copy(
                stage, out_hbm.at[pl.ds(rs, RPT), pl.ds(c * dh, dh)])

    scratch = [
        pltpu.VMEM_SHARED((N, dh), jnp.float32),
        pltpu.VMEM_SHARED((NPAD, dh), jnp.float32),
        pltpu.VMEM((w, CHUNK), jnp.int32),
        pltpu.VMEM((w, CHUNK), jnp.int32),
        pltpu.VMEM((nbuf, CHUNK, dh), jnp.float32),
        pltpu.SemaphoreType.DMA((nbuf,)),
        pltpu.SemaphoreType.DMA((nbuf,)),
    ]
    if finalize:
        scratch += [
            pltpu.VMEM((RPT_PAD,), jnp.float32),
            pltpu.VMEM((dh,), jnp.float32),
            pltpu.VMEM((RPT, dh), jnp.float32),
        ]
    out_shape = (N, NC * dh) if finalize else (NC, N, dh)
    return functools.partial(
        pl.kernel,
        out_type=jax.ShapeDtypeStruct(out_shape, jnp.float32),
        mesh=_mesh,
        scratch_types=scratch,
        compiler_params=pltpu.CompilerParams(needs_layout_passes=False, use_tc_tiling_on_sc=False),
    )(agg_body)


def _make_agg_packed(dh, nbuf, lead, ph):
    w = CPT // ph
    assert w % nbuf == 0 and lead < nbuf

    def agg_body(hs_hbm, pk_hbm, out_hbm,
                 hs_sh, acc_sh, pk_v, usrc, udst, buf, sem, ssem):
        c = lax.axis_index("c")
        s = lax.axis_index("s")
        rs = s * RPT

        pltpu.sync_copy(hs_hbm.at[c, pl.ds(rs, RPT)], hs_sh.at[pl.ds(rs, RPT)])
        pltpu.sync_copy(hs_hbm.at[c, pl.ds(rs, RPT)], acc_sh.at[pl.ds(rs, RPT)])
        plsc.subcore_barrier()

        def unpack(j, slot):
            for k in range(CHUNK // LANES):
                pvec = pk_v[j, pl.ds(k * LANES, LANES)]
                usrc[slot, pl.ds(k * LANES, LANES)] = pvec & 0x3FFF
                udst[slot, pl.ds(k * LANES, LANES)] = (
                    lax.shift_right_logical(pvec, 14))

        for p in range(ph):
            pltpu.sync_copy(pk_hbm.at[s, pl.ds(p * w, w)], pk_v)

            for b in range(lead):
                unpack(b, b)
                pltpu.async_copy(hs_sh.at[usrc.at[b]], buf.at[b], sem.at[b])

            def outer_body(i, carry):
                j0 = i * nbuf
                for b in range(nbuf):
                    j = j0 + b
                    pltpu.make_async_copy(
                        hs_sh.at[usrc.at[b]], buf.at[b], sem.at[b]).wait()
                    pltpu.async_copy(buf.at[b], acc_sh.at[udst.at[b]],
                                     ssem.at[b], add=True)
                    f = j + lead
                    bf = (b + lead) % nbuf

                    @pl.when(f < w)
                    def _():
                        @pl.when(j >= nbuf - lead)
                        def _():
                            pltpu.make_async_copy(
                                buf.at[bf], acc_sh.at[udst.at[bf]],
                                ssem.at[bf]).wait()

                        unpack(f, bf)
                        pltpu.async_copy(hs_sh.at[usrc.at[bf]], buf.at[bf],
                                         sem.at[bf])
                return carry

            lax.fori_loop(0, w // nbuf, outer_body, 0)

            for b in range(nbuf):
                pltpu.make_async_copy(
                    buf.at[b], acc_sh.at[udst.at[b]], ssem.at[b]).wait()

        plsc.subcore_barrier()
        pltpu.sync_copy(acc_sh.at[pl.ds(rs, RPT)],
                        out_hbm.at[c, pl.ds(rs, RPT)])

    return functools.partial(
        pl.kernel,
        out_type=jax.ShapeDtypeStruct((NC, N, dh), jnp.float32),
        mesh=_mesh,
        scratch_types=[
            pltpu.VMEM_SHARED((N, dh), jnp.float32),
            pltpu.VMEM_SHARED((NPAD, dh), jnp.float32),
            pltpu.VMEM((w, CHUNK), jnp.int32),
            pltpu.VMEM((nbuf, CHUNK), jnp.int32),
            pltpu.VMEM((nbuf, CHUNK), jnp.int32),
            pltpu.VMEM((nbuf, CHUNK, dh), jnp.float32),
            pltpu.SemaphoreType.DMA((nbuf,)),
            pltpu.SemaphoreType.DMA((nbuf,)),
        ],
        compiler_params=pltpu.CompilerParams(needs_layout_passes=False, use_tc_tiling_on_sc=False),
    )(agg_body)


_agg_hid = _make_agg_packed(D_HID // NC, nbuf=5, lead=2, ph=4)
_agg_out = _make_agg(D_OUT // NC, nbuf=8, lead=4, ph=2, finalize=True)


def _prep_body(degt_ref, x_ref, w1_ref, hs_ref, dis_ref):
    deg = jnp.sum(degt_ref[...], axis=1, keepdims=True) + 1.0
    dis = lax.rsqrt(deg)
    h = jnp.dot(x_ref[...], w1_ref[...], preferred_element_type=jnp.float32)
    hs = h * dis
    hs_ref[0] = hs[:, : D_HID // 2]
    hs_ref[1] = hs[:, D_HID // 2:]
    dis_ref[...] = dis


def _mid_body(agg_ref, dis_ref, b1_ref, w2_ref, out_ref):
    dis = dis_ref[...]
    h0 = jnp.maximum(agg_ref[0] * dis + b1_ref[0, : D_HID // 2], 0.0)
    h1 = jnp.maximum(agg_ref[1] * dis + b1_ref[0, D_HID // 2:], 0.0)
    hs2 = jnp.dot(h0, w2_ref[: D_HID // 2], preferred_element_type=jnp.float32)
    hs2 = hs2 + jnp.dot(h1, w2_ref[D_HID // 2:], preferred_element_type=jnp.float32)
    hs2 = hs2 * dis
    out_ref[0] = hs2[:, : D_OUT // 2]
    out_ref[1] = hs2[:, D_OUT // 2:]


_prep = pl.pallas_call(
    _prep_body,
    out_shape=[
        jax.ShapeDtypeStruct((NC, N, D_HID // 2), jnp.float32),
        jax.ShapeDtypeStruct((N, 1), jnp.float32),
    ],
)

_mid = pl.pallas_call(
    _mid_body,
    out_shape=jax.ShapeDtypeStruct((NC, N, D_OUT // 2), jnp.float32),
)

@jax.jit
def kernel(x, edge_index, W1, b1, W2, b2):
    src = edge_index[0].astype(jnp.int32)
    dst = edge_index[1].astype(jnp.int32)
    pad = NS * EPT - E
    src_p = jnp.concatenate([src, jnp.zeros((pad,), jnp.int32)]).reshape(NS, CPT, CHUNK)
    dst_p = jnp.concatenate([dst, jnp.full((pad,), N, jnp.int32)]).reshape(NS, CPT, CHUNK)
    pk_p = src_p | (dst_p << 14)

    deg_parts = _deg(dst_p)
    degt = deg_parts.T[:N]
    hs1, dis = _prep(degt, x, W1)
    agg1 = _agg_hid(hs1, pk_p)
    hs2 = _mid(agg1, dis, b1.reshape(1, -1), W2)
    dis_t = jnp.pad(dis.reshape(NS, RPT), ((0, 0), (0, RPT_PAD - RPT)))
    b2_t = b2.reshape(NC, D_OUT // NC)
    return _agg_out(hs2, src_p, dst_p, dis_t, b2_t)


# speedup vs baseline: 1.0332x; 1.0013x over previous
import functools

import jax
import jax.numpy as jnp
from jax import lax
from jax.experimental import pallas as pl
from jax.experimental.pallas import tpu as pltpu
from jax.experimental.pallas import tpu_sc as plsc

N = 10000
E = 320000
D_IN = 128
D_HID = 128
D_OUT = 64

NC = 2
NS = 16
LANES = 16
CHUNK = 128
CPT = 160
EPT = CPT * CHUNK
NPAD = 10016
RPT = N // NS
RPT_PAD = 640

_mesh = plsc.VectorSubcoreMesh(core_axis_name="c", subcore_axis_name="s")


def _deg_body(dst_hbm, out_hbm, deg_v, dst_v):
    c = lax.axis_index("c")
    s = lax.axis_index("s")

    zeros16 = jnp.zeros((LANES,), jnp.float32)

    def zero_body(i, carry):
        deg_v[pl.ds(i * LANES, LANES)] = zeros16
        return carry

    lax.fori_loop(0, NPAD // LANES, zero_body, 0)

    pltpu.sync_copy(dst_hbm.at[s], dst_v)

    ones16 = jnp.ones((LANES,), jnp.float32)
    half = CPT // 2

    def edge_body(j, carry):
        for k in range(CHUNK // LANES):
            idx = dst_v[j, pl.ds(k * LANES, LANES)]
            plsc.addupdate_scatter(deg_v, [idx], ones16)
        return carry

    lax.fori_loop(c * half, (c + 1) * half, edge_body, 0)

    wid = s * NC + c
    pltpu.sync_copy(deg_v, out_hbm.at[wid])


_deg = functools.partial(
    pl.kernel,
    out_type=jax.ShapeDtypeStruct((NC * NS, NPAD), jnp.float32),
    mesh=_mesh,
    scratch_types=[
        pltpu.VMEM((NPAD,), jnp.float32),
        pltpu.VMEM((CPT, CHUNK), jnp.int32),
    ],
    compiler_params=pltpu.CompilerParams(needs_layout_passes=False, use_tc_tiling_on_sc=False),
)(_deg_body)


def _make_agg(dh, nbuf, lead, ph, finalize=False):
    w = CPT // ph
    assert w % nbuf == 0 and lead < nbuf

    def agg_body(hs_hbm, src_hbm, dst_hbm, *rest):
        if finalize:
            (dis_hbm, b_hbm, out_hbm, hs_sh, acc_sh, src_v, dst_v, buf,
             sem, ssem, dis_v, b_v, stage) = rest
        else:
            (out_hbm, hs_sh, acc_sh, src_v, dst_v, buf, sem, ssem) = rest
        c = lax.axis_index("c")
        s = lax.axis_index("s")
        rs = s * RPT

        pltpu.sync_copy(hs_hbm.at[c, pl.ds(rs, RPT)], hs_sh.at[pl.ds(rs, RPT)])
        pltpu.sync_copy(hs_hbm.at[c, pl.ds(rs, RPT)], acc_sh.at[pl.ds(rs, RPT)])
        plsc.subcore_barrier()

        for p in range(ph):
            pltpu.sync_copy(src_hbm.at[s, pl.ds(p * w, w)], src_v)
            pltpu.sync_copy(dst_hbm.at[s, pl.ds(p * w, w)], dst_v)

            for b in range(lead):
                pltpu.async_copy(hs_sh.at[src_v.at[b]], buf.at[b],
                                 sem.at[b])

            def outer_body(i, carry):
                j0 = i * nbuf
                for b in range(nbuf):
                    j = j0 + b
                    pltpu.make_async_copy(
                        hs_sh.at[src_v.at[j]], buf.at[b], sem.at[b]).wait()
                    pltpu.async_copy(buf.at[b], acc_sh.at[dst_v.at[j]],
                                     ssem.at[b], add=True)
                    f = j + lead
                    bf = (b + lead) % nbuf

                    @pl.when(f < w)
                    def _():
                        @pl.when(j >= nbuf - lead)
                        def _():
                            pltpu.make_async_copy(
                                buf.at[bf], acc_sh.at[dst_v.at[j]],
                                ssem.at[bf]).wait()

                        pltpu.async_copy(hs_sh.at[src_v.at[f]], buf.at[bf],
                                         sem.at[bf])
                return carry

            lax.fori_loop(0, w // nbuf, outer_body, 0)

            for b in range(nbuf):
                pltpu.make_async_copy(
                    buf.at[b], acc_sh.at[dst_v.at[w - nbuf + b]],
                    ssem.at[b]).wait()

        plsc.subcore_barrier()

        if not finalize:
            pltpu.sync_copy(acc_sh.at[pl.ds(rs, RPT)],
                            out_hbm.at[c, pl.ds(rs, RPT)])
        else:
            pltpu.sync_copy(acc_sh.at[pl.ds(rs, RPT)], stage)
            pltpu.sync_copy(dis_hbm.at[s], dis_v)
            pltpu.sync_copy(b_hbm.at[c], b_v)
            bias_vecs = [b_v[pl.ds(k * LANES, LANES)]
                         for k in range(dh // LANES)]

            def row_body(r, carry):
                d = plsc.load_gather(dis_v, [jnp.zeros((LANES,), jnp.int32) + r])
                for k in range(dh // LANES):
                    v = stage[r, pl.ds(k * LANES, LANES)]
                    stage[r, pl.ds(k * LANES, LANES)] = v * d + bias_vecs[k]
                return carry

            lax.fori_loop(0, RPT, row_body, 0)
            pltpu.sync_copy(
                stage, out_hbm.at[pl.ds(rs, RPT), pl.ds(c * dh, dh)])

    scratch = [
        pltpu.VMEM_SHARED((N, dh), jnp.float32),
        pltpu.VMEM_SHARED((NPAD, dh), jnp.float32),
        pltpu.VMEM((w, CHUNK), jnp.int32),
        pltpu.VMEM((w, CHUNK), jnp.int32),
        pltpu.VMEM((nbuf, CHUNK, dh), jnp.float32),
        pltpu.SemaphoreType.DMA((nbuf,)),
        pltpu.SemaphoreType.DMA((nbuf,)),
    ]
    if finalize:
        scratch += [
            pltpu.VMEM((RPT_PAD,), jnp.float32),
            pltpu.VMEM((dh,), jnp.float32),
            pltpu.VMEM((RPT, dh), jnp.float32),
        ]
    out_shape = (N, NC * dh) if finalize else (NC, N, dh)
    return functools.partial(
        pl.kernel,
        out_type=jax.ShapeDtypeStruct(out_shape, jnp.float32),
        mesh=_mesh,
        scratch_types=scratch,
        compiler_params=pltpu.CompilerParams(needs_layout_passes=False, use_tc_tiling_on_sc=False),
    )(agg_body)


def _make_agg_packed(dh, nbuf, lead, ph):
    w = CPT // ph
    assert w % nbuf == 0 and lead < nbuf

    def agg_body(hs_hbm, pk_hbm, out_hbm,
                 hs_sh, acc_sh, pk_v, usrc, udst, buf, sem, ssem):
        c = lax.axis_index("c")
        s = lax.axis_index("s")
        rs = s * RPT

        pltpu.sync_copy(hs_hbm.at[c, pl.ds(rs, RPT)], hs_sh.at[pl.ds(rs, RPT)])
        pltpu.sync_copy(hs_hbm.at[c, pl.ds(rs, RPT)], acc_sh.at[pl.ds(rs, RPT)])
        plsc.subcore_barrier()

        def unpack(j, slot):
            for k in range(CHUNK // LANES):
                pvec = pk_v[j, pl.ds(k * LANES, LANES)]
                usrc[slot, pl.ds(k * LANES, LANES)] = pvec & 0x3FFF
                udst[slot, pl.ds(k * LANES, LANES)] = (
                    lax.shift_right_logical(pvec, 14))

        for p in range(ph):
            pltpu.sync_copy(pk_hbm.at[s, pl.ds(p * w, w)], pk_v)

            for b in range(lead):
                unpack(b, b)
                pltpu.async_copy(hs_sh.at[usrc.at[b]], buf.at[b], sem.at[b])

            def outer_body(i, carry):
                j0 = i * nbuf
                for b in range(nbuf):
                    j = j0 + b
                    pltpu.make_async_copy(
                        hs_sh.at[usrc.at[b]], buf.at[b], sem.at[b]).wait()
                    pltpu.async_copy(buf.at[b], acc_sh.at[udst.at[b]],
                                     ssem.at[b], add=True)
                    f = j + lead
                    bf = (b + lead) % nbuf

                    @pl.when(f < w)
                    def _():
                        @pl.when(j >= nbuf - lead)
                        def _():
                            pltpu.make_async_copy(
                                buf.at[bf], acc_sh.at[udst.at[bf]],
                                ssem.at[bf]).wait()

                        unpack(f, bf)
                        pltpu.async_copy(hs_sh.at[usrc.at[bf]], buf.at[bf],
                                         sem.at[bf])
                return carry

            lax.fori_loop(0, w // nbuf, outer_body, 0)

            for b in range(nbuf):
                pltpu.make_async_copy(
                    buf.at[b], acc_sh.at[udst.at[b]], ssem.at[b]).wait()

        plsc.subcore_barrier()
        pltpu.sync_copy(acc_sh.at[pl.ds(rs, RPT)],
                        out_hbm.at[c, pl.ds(rs, RPT)])

    return functools.partial(
        pl.kernel,
        out_type=jax.ShapeDtypeStruct((NC, N, dh), jnp.float32),
        mesh=_mesh,
        scratch_types=[
            pltpu.VMEM_SHARED((N, dh), jnp.float32),
            pltpu.VMEM_SHARED((NPAD, dh), jnp.float32),
            pltpu.VMEM((w, CHUNK), jnp.int32),
            pltpu.VMEM((nbuf, CHUNK), jnp.int32),
            pltpu.VMEM((nbuf, CHUNK), jnp.int32),
            pltpu.VMEM((nbuf, CHUNK, dh), jnp.float32),
            pltpu.SemaphoreType.DMA((nbuf,)),
            pltpu.SemaphoreType.DMA((nbuf,)),
        ],
        compiler_params=pltpu.CompilerParams(needs_layout_passes=False, use_tc_tiling_on_sc=False),
    )(agg_body)


_agg_hid = _make_agg_packed(D_HID // NC, nbuf=5, lead=3, ph=4)
_agg_out = _make_agg(D_OUT // NC, nbuf=8, lead=4, ph=2, finalize=True)


def _prep_body(degt_ref, x_ref, w1_ref, hs_ref, dis_ref):
    deg = jnp.sum(degt_ref[...], axis=1, keepdims=True) + 1.0
    dis = lax.rsqrt(deg)
    h = jnp.dot(x_ref[...], w1_ref[...], preferred_element_type=jnp.float32)
    hs = h * dis
    hs_ref[0] = hs[:, : D_HID // 2]
    hs_ref[1] = hs[:, D_HID // 2:]
    dis_ref[...] = dis


def _mid_body(agg_ref, dis_ref, b1_ref, w2_ref, out_ref):
    dis = dis_ref[...]
    h0 = jnp.maximum(agg_ref[0] * dis + b1_ref[0, : D_HID // 2], 0.0)
    h1 = jnp.maximum(agg_ref[1] * dis + b1_ref[0, D_HID // 2:], 0.0)
    hs2 = jnp.dot(h0, w2_ref[: D_HID // 2], preferred_element_type=jnp.float32)
    hs2 = hs2 + jnp.dot(h1, w2_ref[D_HID // 2:], preferred_element_type=jnp.float32)
    hs2 = hs2 * dis
    out_ref[0] = hs2[:, : D_OUT // 2]
    out_ref[1] = hs2[:, D_OUT // 2:]


_prep = pl.pallas_call(
    _prep_body,
    out_shape=[
        jax.ShapeDtypeStruct((NC, N, D_HID // 2), jnp.float32),
        jax.ShapeDtypeStruct((N, 1), jnp.float32),
    ],
)

_mid = pl.pallas_call(
    _mid_body,
    out_shape=jax.ShapeDtypeStruct((NC, N, D_OUT // 2), jnp.float32),
)

@jax.jit
def kernel(x, edge_index, W1, b1, W2, b2):
    src = edge_index[0].astype(jnp.int32)
    dst = edge_index[1].astype(jnp.int32)
    pad = NS * EPT - E
    src_p = jnp.concatenate([src, jnp.zeros((pad,), jnp.int32)]).reshape(NS, CPT, CHUNK)
    dst_p = jnp.concatenate([dst, jnp.full((pad,), N, jnp.int32)]).reshape(NS, CPT, CHUNK)
    pk_p = src_p | (dst_p << 14)

    deg_parts = _deg(dst_p)
    degt = deg_parts.T[:N]
    hs1, dis = _prep(degt, x, W1)
    agg1 = _agg_hid(hs1, pk_p)
    hs2 = _mid(agg1, dis, b1.reshape(1, -1), W2)
    dis_t = jnp.pad(dis.reshape(NS, RPT), ((0, 0), (0, RPT_PAD - RPT)))
    b2_t = b2.reshape(NC, D_OUT // NC)
    return _agg_out(hs2, src_p, dst_p, dis_t, b2_t)


# speedup vs baseline: 1.0411x; 1.0076x over previous
import functools

import jax
import jax.numpy as jnp
from jax import lax
from jax.experimental import pallas as pl
from jax.experimental.pallas import tpu as pltpu
from jax.experimental.pallas import tpu_sc as plsc

N = 10000
E = 320000
D_IN = 128
D_HID = 128
D_OUT = 64

NC = 2
NS = 16
LANES = 16
CHUNK = 128
CPT = 160
EPT = CPT * CHUNK
NPAD = 10016
RPT = N // NS
RPT_PAD = 640

_mesh = plsc.VectorSubcoreMesh(core_axis_name="c", subcore_axis_name="s")


def _deg_body(dst_hbm, out_hbm, deg_v, dst_v):
    c = lax.axis_index("c")
    s = lax.axis_index("s")

    zeros16 = jnp.zeros((LANES,), jnp.float32)

    def zero_body(i, carry):
        deg_v[pl.ds(i * LANES, LANES)] = zeros16
        return carry

    lax.fori_loop(0, NPAD // LANES, zero_body, 0)

    pltpu.sync_copy(dst_hbm.at[s], dst_v)

    ones16 = jnp.ones((LANES,), jnp.float32)
    half = CPT // 2

    def edge_body(j, carry):
        for k in range(CHUNK // LANES):
            idx = dst_v[j, pl.ds(k * LANES, LANES)]
            plsc.addupdate_scatter(deg_v, [idx], ones16)
        return carry

    lax.fori_loop(c * half, (c + 1) * half, edge_body, 0)

    wid = s * NC + c
    pltpu.sync_copy(deg_v, out_hbm.at[wid])


_deg = functools.partial(
    pl.kernel,
    out_type=jax.ShapeDtypeStruct((NC * NS, NPAD), jnp.float32),
    mesh=_mesh,
    scratch_types=[
        pltpu.VMEM((NPAD,), jnp.float32),
        pltpu.VMEM((CPT, CHUNK), jnp.int32),
    ],
    compiler_params=pltpu.CompilerParams(needs_layout_passes=False, use_tc_tiling_on_sc=False),
)(_deg_body)


def _make_agg(dh, nbuf, lead, ph, finalize=False):
    w = CPT // ph
    assert w % nbuf == 0 and lead < nbuf

    def agg_body(hs_hbm, src_hbm, dst_hbm, *rest):
        if finalize:
            (dis_hbm, b_hbm, out_hbm, hs_sh, acc_sh, src_v, dst_v, buf,
             sem, ssem, dis_v, b_v, stage) = rest
        else:
            (out_hbm, hs_sh, acc_sh, src_v, dst_v, buf, sem, ssem) = rest
        c = lax.axis_index("c")
        s = lax.axis_index("s")
        rs = s * RPT

        pltpu.sync_copy(hs_hbm.at[c, pl.ds(rs, RPT)], hs_sh.at[pl.ds(rs, RPT)])
        pltpu.sync_copy(hs_hbm.at[c, pl.ds(rs, RPT)], acc_sh.at[pl.ds(rs, RPT)])
        plsc.subcore_barrier()

        for p in range(ph):
            pltpu.sync_copy(src_hbm.at[s, pl.ds(p * w, w)], src_v)
            pltpu.sync_copy(dst_hbm.at[s, pl.ds(p * w, w)], dst_v)

            for b in range(lead):
                pltpu.async_copy(hs_sh.at[src_v.at[b]], buf.at[b],
                                 sem.at[b])

            def outer_body(i, carry):
                j0 = i * nbuf
                for b in range(nbuf):
                    j = j0 + b
                    pltpu.make_async_copy(
                        hs_sh.at[src_v.at[j]], buf.at[b], sem.at[b]).wait()
                    pltpu.async_copy(buf.at[b], acc_sh.at[dst_v.at[j]],
                                     ssem.at[b], add=True)
                    f = j + lead
                    bf = (b + lead) % nbuf

                    @pl.when(f < w)
                    def _():
                        @pl.when(j >= nbuf - lead)
                        def _():
                            pltpu.make_async_copy(
                                buf.at[bf], acc_sh.at[dst_v.at[j]],
                                ssem.at[bf]).wait()

                        pltpu.async_copy(hs_sh.at[src_v.at[f]], buf.at[bf],
                                         sem.at[bf])
                return carry

            lax.fori_loop(0, w // nbuf, outer_body, 0)

            for b in range(nbuf):
                pltpu.make_async_copy(
                    buf.at[b], acc_sh.at[dst_v.at[w - nbuf + b]],
                    ssem.at[b]).wait()

        plsc.subcore_barrier()

        if not finalize:
            pltpu.sync_copy(acc_sh.at[pl.ds(rs, RPT)],
                            out_hbm.at[c, pl.ds(rs, RPT)])
        else:
            pltpu.sync_copy(acc_sh.at[pl.ds(rs, RPT)], stage)
            pltpu.sync_copy(dis_hbm.at[s], dis_v)
            pltpu.sync_copy(b_hbm.at[c], b_v)
            bias_vecs = [b_v[pl.ds(k * LANES, LANES)]
                         for k in range(dh // LANES)]

            def row_body(r, carry):
                d = plsc.load_gather(dis_v, [jnp.zeros((LANES,), jnp.int32) + r])
                for k in range(dh // LANES):
                    v = stage[r, pl.ds(k * LANES, LANES)]
                    stage[r, pl.ds(k * LANES, LANES)] = v * d + bias_vecs[k]
                return carry

            lax.fori_loop(0, RPT, row_body, 0)
            pltpu.sync_copy(
                stage, out_hbm.at[pl.ds(rs, RPT), pl.ds(c * dh, dh)])

    scratch = [
        pltpu.VMEM_SHARED((N, dh), jnp.float32),
        pltpu.VMEM_SHARED((NPAD, dh), jnp.float32),
        pltpu.VMEM((w, CHUNK), jnp.int32),
        pltpu.VMEM((w, CHUNK), jnp.int32),
        pltpu.VMEM((nbuf, CHUNK, dh), jnp.float32),
        pltpu.SemaphoreType.DMA((nbuf,)),
        pltpu.SemaphoreType.DMA((nbuf,)),
    ]
    if finalize:
        scratch += [
            pltpu.VMEM((RPT_PAD,), jnp.float32),
            pltpu.VMEM((dh,), jnp.float32),
            pltpu.VMEM((RPT, dh), jnp.float32),
        ]
    out_shape = (N, NC * dh) if finalize else (NC, N, dh)
    return functools.partial(
        pl.kernel,
        out_type=jax.ShapeDtypeStruct(out_shape, jnp.float32),
        mesh=_mesh,
        scratch_types=scratch,
        compiler_params=pltpu.CompilerParams(needs_layout_passes=False, use_tc_tiling_on_sc=False),
    )(agg_body)


def _make_agg_packed(dh, nbuf, lead, ph, finalize=False):
    w = CPT // ph
    assert w % nbuf == 0 and lead < nbuf

    def agg_body(hs_hbm, pk_hbm, *rest):
        if finalize:
            (dis_hbm, b_hbm, out_hbm, hs_sh, acc_sh, pk_v, usrc, udst,
             buf, sem, ssem, dis_v, b_v, stage) = rest
        else:
            (out_hbm, hs_sh, acc_sh, pk_v, usrc, udst, buf, sem, ssem) = rest
        c = lax.axis_index("c")
        s = lax.axis_index("s")
        rs = s * RPT

        pltpu.sync_copy(hs_hbm.at[c, pl.ds(rs, RPT)], hs_sh.at[pl.ds(rs, RPT)])
        pltpu.sync_copy(hs_hbm.at[c, pl.ds(rs, RPT)], acc_sh.at[pl.ds(rs, RPT)])
        plsc.subcore_barrier()

        def unpack(j, slot):
            for k in range(CHUNK // LANES):
                pvec = pk_v[j, pl.ds(k * LANES, LANES)]
                usrc[slot, pl.ds(k * LANES, LANES)] = pvec & 0x3FFF
                udst[slot, pl.ds(k * LANES, LANES)] = (
                    lax.shift_right_logical(pvec, 14))

        for p in range(ph):
            pltpu.sync_copy(pk_hbm.at[s, pl.ds(p * w, w)], pk_v)

            for b in range(lead):
                unpack(b, b)
                pltpu.async_copy(hs_sh.at[usrc.at[b]], buf.at[b], sem.at[b])

            def outer_body(i, carry):
                j0 = i * nbuf
                for b in range(nbuf):
                    j = j0 + b
                    pltpu.make_async_copy(
                        hs_sh.at[usrc.at[b]], buf.at[b], sem.at[b]).wait()
                    pltpu.async_copy(buf.at[b], acc_sh.at[udst.at[b]],
                                     ssem.at[b], add=True)
                    f = j + lead
                    bf = (b + lead) % nbuf

                    @pl.when(f < w)
                    def _():
                        @pl.when(j >= nbuf - lead)
                        def _():
                            pltpu.make_async_copy(
                                buf.at[bf], acc_sh.at[udst.at[bf]],
                                ssem.at[bf]).wait()

                        unpack(f, bf)
                        pltpu.async_copy(hs_sh.at[usrc.at[bf]], buf.at[bf],
                                         sem.at[bf])
                return carry

            lax.fori_loop(0, w // nbuf, outer_body, 0)

            for b in range(nbuf):
                pltpu.make_async_copy(
                    buf.at[b], acc_sh.at[udst.at[b]], ssem.at[b]).wait()

        plsc.subcore_barrier()

        if not finalize:
            pltpu.sync_copy(acc_sh.at[pl.ds(rs, RPT)],
                            out_hbm.at[c, pl.ds(rs, RPT)])
        else:
            pltpu.sync_copy(acc_sh.at[pl.ds(rs, RPT)], stage)
            pltpu.sync_copy(dis_hbm.at[s], dis_v)
            pltpu.sync_copy(b_hbm.at[c], b_v)
            bias_vecs = [b_v[pl.ds(k * LANES, LANES)]
                         for k in range(dh // LANES)]

            def row_body(r, carry):
                d = plsc.load_gather(dis_v,
                                     [jnp.zeros((LANES,), jnp.int32) + r])
                for k in range(dh // LANES):
                    v = stage[r, pl.ds(k * LANES, LANES)]
                    stage[r, pl.ds(k * LANES, LANES)] = v * d + bias_vecs[k]
                return carry

            lax.fori_loop(0, RPT, row_body, 0)
            pltpu.sync_copy(
                stage, out_hbm.at[pl.ds(rs, RPT), pl.ds(c * dh, dh)])

    scratch = [
        pltpu.VMEM_SHARED((N, dh), jnp.float32),
        pltpu.VMEM_SHARED((NPAD, dh), jnp.float32),
        pltpu.VMEM((w, CHUNK), jnp.int32),
        pltpu.VMEM((nbuf, CHUNK), jnp.int32),
        pltpu.VMEM((nbuf, CHUNK), jnp.int32),
        pltpu.VMEM((nbuf, CHUNK, dh), jnp.float32),
        pltpu.SemaphoreType.DMA((nbuf,)),
        pltpu.SemaphoreType.DMA((nbuf,)),
    ]
    if finalize:
        scratch += [
            pltpu.VMEM((RPT_PAD,), jnp.float32),
            pltpu.VMEM((dh,), jnp.float32),
            pltpu.VMEM((RPT, dh), jnp.float32),
        ]
    out_shape = (N, NC * dh) if finalize else (NC, N, dh)
    return functools.partial(
        pl.kernel,
        out_type=jax.ShapeDtypeStruct(out_shape, jnp.float32),
        mesh=_mesh,
        scratch_types=scratch,
        compiler_params=pltpu.CompilerParams(needs_layout_passes=False, use_tc_tiling_on_sc=False),
    )(agg_body)


_agg_hid = _make_agg_packed(D_HID // NC, nbuf=5, lead=3, ph=4)
_agg_out = _make_agg_packed(D_OUT // NC, nbuf=10, lead=5, ph=1, finalize=True)


def _prep_body(degt_ref, x_ref, w1_ref, hs_ref, dis_ref):
    deg = jnp.sum(degt_ref[...], axis=1, keepdims=True) + 1.0
    dis = lax.rsqrt(deg)
    h = jnp.dot(x_ref[...], w1_ref[...], preferred_element_type=jnp.float32)
    hs = h * dis
    hs_ref[0] = hs[:, : D_HID // 2]
    hs_ref[1] = hs[:, D_HID // 2:]
    dis_ref[...] = dis


def _mid_body(agg_ref, dis_ref, b1_ref, w2_ref, out_ref):
    dis = dis_ref[...]
    h0 = jnp.maximum(agg_ref[0] * dis + b1_ref[0, : D_HID // 2], 0.0)
    h1 = jnp.maximum(agg_ref[1] * dis + b1_ref[0, D_HID // 2:], 0.0)
    hs2 = jnp.dot(h0, w2_ref[: D_HID // 2], preferred_element_type=jnp.float32)
    hs2 = hs2 + jnp.dot(h1, w2_ref[D_HID // 2:], preferred_element_type=jnp.float32)
    hs2 = hs2 * dis
    out_ref[0] = hs2[:, : D_OUT // 2]
    out_ref[1] = hs2[:, D_OUT // 2:]


_prep = pl.pallas_call(
    _prep_body,
    out_shape=[
        jax.ShapeDtypeStruct((NC, N, D_HID // 2), jnp.float32),
        jax.ShapeDtypeStruct((N, 1), jnp.float32),
    ],
)

_mid = pl.pallas_call(
    _mid_body,
    out_shape=jax.ShapeDtypeStruct((NC, N, D_OUT // 2), jnp.float32),
)

@jax.jit
def kernel(x, edge_index, W1, b1, W2, b2):
    src = edge_index[0].astype(jnp.int32)
    dst = edge_index[1].astype(jnp.int32)
    pad = NS * EPT - E
    src_p = jnp.concatenate([src, jnp.zeros((pad,), jnp.int32)]).reshape(NS, CPT, CHUNK)
    dst_p = jnp.concatenate([dst, jnp.full((pad,), N, jnp.int32)]).reshape(NS, CPT, CHUNK)
    pk_p = src_p | (dst_p << 14)

    deg_parts = _deg(dst_p)
    degt = deg_parts.T[:N]
    hs1, dis = _prep(degt, x, W1)
    agg1 = _agg_hid(hs1, pk_p)
    hs2 = _mid(agg1, dis, b1.reshape(1, -1), W2)
    dis_t = jnp.pad(dis.reshape(NS, RPT), ((0, 0), (0, RPT_PAD - RPT)))
    b2_t = b2.reshape(NC, D_OUT // NC)
    return _agg_out(hs2, pk_p, dis_t, b2_t)
